# baseline (speedup 1.0000x reference)
import functools

import jax
import jax.numpy as jnp
from jax import lax
from jax.experimental import pallas as pl
from jax.experimental.pallas import tpu as pltpu
from jax.experimental.pallas import tpu_sc as plsc

B, T, D = 16, 512, 32
L = 16
NC = 2
NS = 16


def _impute_body(x_hbm, out_hbm, xv, piv, pav, ov):
    c = lax.axis_index("c")
    s = lax.axis_index("s")
    w = s * NC + c
    b = w // 2
    h = (w % 2) * L

    pltpu.sync_copy(x_hbm.at[b, :, pl.ds(h, L)], xv)

    zero_f = jnp.zeros((L,), jnp.float32)

    def fwd(t, carry):
        pidx, pval = carry
        xt = xv[t]
        tvec = jnp.full((L,), t, jnp.int32)
        m = xt != 0.0
        pidx = jnp.where(m, tvec, pidx)
        pval = jnp.where(m, xt, pval)
        piv[t] = pidx
        pav[t] = pval
        return pidx, pval

    lax.fori_loop(0, T, fwd, (jnp.full((L,), -1, jnp.int32), zero_f),
                  unroll=8)

    def bwd(i, carry):
        nidx, nval = carry
        t = T - 1 - i
        xt = xv[t]
        tvec = jnp.full((L,), t, jnp.int32)
        m = xt != 0.0
        nidx = jnp.where(m, tvec, nidx)
        nval = jnp.where(m, xt, nval)
        pidx = piv[t]
        pval = pav[t]
        start = jnp.maximum(pidx, 0)
        end = jnp.minimum(nidx, T - 1)
        denom = jnp.maximum(end - start - 1, 1).astype(jnp.float32)
        interp = pval + (tvec - start).astype(jnp.float32) * (nval - pval) / denom
        ov[t] = jnp.where(m, xt, jnp.where(tvec < end, interp, xt))
        return nidx, nval

    lax.fori_loop(0, T, bwd, (jnp.full((L,), T, jnp.int32), zero_f),
                  unroll=8)

    pltpu.sync_copy(ov, out_hbm.at[b, :, pl.ds(h, L)])


_impute = pl.kernel(
    _impute_body,
    mesh=plsc.VectorSubcoreMesh(core_axis_name="c", subcore_axis_name="s"),
    compiler_params=pltpu.CompilerParams(use_tc_tiling_on_sc=False),
    out_type=jax.ShapeDtypeStruct((B, T, D), jnp.float32),
    scratch_types=[
        pltpu.VMEM((T, L), jnp.float32),
        pltpu.VMEM((T, L), jnp.int32),
        pltpu.VMEM((T, L), jnp.float32),
        pltpu.VMEM((T, L), jnp.float32),
    ],
)


def kernel(x_masked):
    return _impute(x_masked)


# speedup vs baseline: 1.0183x; 1.0183x over previous
import functools

import jax
import jax.numpy as jnp
from jax import lax
from jax.experimental import pallas as pl
from jax.experimental.pallas import tpu as pltpu
from jax.experimental.pallas import tpu_sc as plsc

B, T, D = 16, 512, 32
L = 16
NC = 2
NS = 16


def _impute_body(x_hbm, out_hbm, xv, dtv, pav, ov):
    c = lax.axis_index("c")
    s = lax.axis_index("s")
    w = s * NC + c
    b = w // 2
    h = (w % 2) * L

    pltpu.sync_copy(x_hbm.at[b, :, pl.ds(h, L)], xv)

    zero_f = jnp.zeros((L,), jnp.float32)

    def fwd(t, carry):
        dt, pval = carry
        xt = xv[t]
        m = xt != 0.0
        dt = jnp.where(m, 0.0, dt + 1.0)
        pval = jnp.where(m, xt, pval)
        dtv[t] = dt
        pav[t] = pval
        return dt, pval

    lax.fori_loop(0, T, fwd, (jnp.full((L,), -1.0, jnp.float32), zero_f),
                  unroll=8)

    def bwd(i, carry):
        du, nval = carry
        t = T - 1 - i
        xt = xv[t]
        m = xt != 0.0
        du = jnp.where(m, 0.0, du + 1.0)
        nval = jnp.where(m, xt, nval)
        dt = dtv[t]
        pval = pav[t]
        denom = jnp.maximum(du + dt - 1.0, 1.0)
        interp = pval + dt * (nval - pval) / denom
        ov[t] = jnp.where(du >= 1.0, interp, xt)
        return du, nval

    lax.fori_loop(0, T, bwd, (jnp.full((L,), -1.0, jnp.float32), zero_f),
                  unroll=8)

    pltpu.sync_copy(ov, out_hbm.at[b, :, pl.ds(h, L)])


_impute = pl.kernel(
    _impute_body,
    mesh=plsc.VectorSubcoreMesh(core_axis_name="c", subcore_axis_name="s"),
    compiler_params=pltpu.CompilerParams(use_tc_tiling_on_sc=False),
    out_type=jax.ShapeDtypeStruct((B, T, D), jnp.float32),
    scratch_types=[
        pltpu.VMEM((T, L), jnp.float32),
        pltpu.VMEM((T, L), jnp.float32),
        pltpu.VMEM((T, L), jnp.float32),
        pltpu.VMEM((T, L), jnp.float32),
    ],
)


def kernel(x_masked):
    return _impute(x_masked)


# speedup vs baseline: 1.0200x; 1.0017x over previous
import functools

import jax
import jax.numpy as jnp
from jax import lax
from jax.experimental import pallas as pl
from jax.experimental.pallas import tpu as pltpu
from jax.experimental.pallas import tpu_sc as plsc

B, T, D = 16, 512, 32
L = 16
NC = 2
NS = 16


def _impute_body(x_hbm, out_hbm, xv, dtv, pav, ov):
    c = lax.axis_index("c")
    s = lax.axis_index("s")
    w = s * NC + c
    b = w // 2
    h = (w % 2) * L

    pltpu.sync_copy(x_hbm.at[b, :, pl.ds(h, L)], xv)

    zero_f = jnp.zeros((L,), jnp.float32)

    def fwd(t, carry):
        dt, pval = carry
        xt = xv[t]
        m = xt != 0.0
        dt = jnp.where(m, 0.0, dt + 1.0)
        pval = jnp.where(m, xt, pval)
        dtv[t] = dt
        pav[t] = pval
        return dt, pval

    lax.fori_loop(0, T, fwd, (jnp.full((L,), -1.0, jnp.float32), zero_f),
                  unroll=8)

    def bwd(i, carry):
        du, nval = carry
        t = T - 1 - i
        xt = xv[t]
        m = xt != 0.0
        du = jnp.where(m, 0.0, du + 1.0)
        nval = jnp.where(m, xt, nval)
        dt = dtv[t]
        pval = pav[t]
        denom = jnp.maximum(du + dt - 1.0, 1.0)
        interp = pval + dt * (nval - pval) / denom
        ov[t] = jnp.where(du >= 1.0, interp, xt)
        return du, nval

    lax.fori_loop(0, T, bwd, (jnp.full((L,), -1.0, jnp.float32), zero_f),
                  unroll=8)

    pltpu.sync_copy(ov, out_hbm.at[b, :, pl.ds(h, L)])


_impute = pl.kernel(
    _impute_body,
    mesh=plsc.VectorSubcoreMesh(core_axis_name="c", subcore_axis_name="s"),
    compiler_params=pltpu.CompilerParams(
        use_tc_tiling_on_sc=False,
        disable_bounds_checks=True,
        disable_semaphore_checks=True,
        skip_device_barrier=True,
    ),
    out_type=jax.ShapeDtypeStruct((B, T, D), jnp.float32),
    scratch_types=[
        pltpu.VMEM((T, L), jnp.float32),
        pltpu.VMEM((T, L), jnp.float32),
        pltpu.VMEM((T, L), jnp.float32),
        pltpu.VMEM((T, L), jnp.float32),
    ],
)


def kernel(x_masked):
    return _impute(x_masked)
